# baseline (speedup 1.0000x reference)
import functools

import jax
import jax.numpy as jnp
from jax import lax
from jax.experimental import pallas as pl
from jax.experimental.pallas import tpu as pltpu
from jax.experimental.pallas import tpu_sc as plsc

E = 800000
D = 64
NUM_ROWS = 5

NC = 2
NS = 16
NW = NC * NS

IDX_MINOR = 125
G = 8
CHUNK = G * IDX_MINOR
ROWS_PER_W = E // NW
CHUNKS_PER_W = ROWS_PER_W // CHUNK
IDX_ROWS_PER_W = ROWS_PER_W // IDX_MINOR


def _embed_body(idx_hbm, table_hbm, out_hbm, idx_v, rows_v, sem):
    wid = lax.axis_index("c") * NS + lax.axis_index("s")
    idx_row_base = wid * IDX_ROWS_PER_W
    out_base = wid * ROWS_PER_W

    def chunk_body(i, carry):
        rbase = idx_row_base + i * G
        obase = out_base + i * CHUNK
        pltpu.sync_copy(idx_hbm.at[pl.ds(rbase, G)], idx_v)
        copies = []
        for j in range(G):
            copies.append(
                pltpu.async_copy(
                    table_hbm.at[idx_v.at[j]],
                    rows_v.at[pl.ds(j * IDX_MINOR, IDX_MINOR)],
                    sem,
                )
            )
        for c in copies:
            c.wait()
        pltpu.sync_copy(rows_v, out_hbm.at[pl.ds(obase, CHUNK)])
        return carry

    lax.fori_loop(0, CHUNKS_PER_W, chunk_body, 0)


def kernel(bond_types, table):
    idx2d = bond_types.reshape(E // IDX_MINOR, IDX_MINOR).astype(jnp.int32)
    mesh = plsc.VectorSubcoreMesh(core_axis_name="c", subcore_axis_name="s")
    kern = functools.partial(
        pl.kernel,
        out_type=jax.ShapeDtypeStruct((E, D), jnp.float32),
        mesh=mesh,
        scratch_types=[
            pltpu.VMEM((G, IDX_MINOR), jnp.int32),
            pltpu.VMEM((CHUNK, D), jnp.float32),
            pltpu.SemaphoreType.DMA,
        ],
        compiler_params=pltpu.CompilerParams(use_tc_tiling_on_sc=False),
    )(_embed_body)
    return kern(idx2d, table)


# speedup vs baseline: 11.1382x; 11.1382x over previous
import functools

import jax
import jax.numpy as jnp
from jax import lax
from jax.experimental import pallas as pl
from jax.experimental.pallas import tpu as pltpu
from jax.experimental.pallas import tpu_sc as plsc

E = 800000
D = 64
NUM_ROWS = 5

NC = 2
NS = 16
NW = NC * NS

IDX_MINOR = 125
G = 8
CHUNK = G * IDX_MINOR
NBUF = 2
ROWS_PER_W = E // NW
CHUNKS_PER_W = ROWS_PER_W // CHUNK
IDX_ROWS_PER_W = ROWS_PER_W // IDX_MINOR


def _embed_body(idx_hbm, table_hbm, out_hbm, table_sh,
                idx0, idx1, rows0, rows1, gsem0, gsem1, ssem0, ssem1):
    cid = lax.axis_index("c")
    sid = lax.axis_index("s")
    wid = cid * NS + sid
    idx_row_base = wid * IDX_ROWS_PER_W
    out_base = wid * ROWS_PER_W

    @pl.when(sid == 0)
    def _():
        pltpu.sync_copy(table_hbm, table_sh)

    plsc.subcore_barrier()

    idx_bufs = (idx0, idx1)
    rows_bufs = (rows0, rows1)
    gsems = (gsem0, gsem1)
    ssems = (ssem0, ssem1)

    def fire_gathers(c, b):
        rbase = idx_row_base + c * G
        pltpu.sync_copy(idx_hbm.at[pl.ds(rbase, G)], idx_bufs[b])
        handles = []
        for j in range(G):
            handles.append(pltpu.async_copy(
                table_sh.at[idx_bufs[b].at[j]],
                rows_bufs[b].at[pl.ds(j * IDX_MINOR, IDX_MINOR)],
                gsems[b],
            ))
        return handles

    def fire_store(c, b):
        obase = out_base + c * CHUNK
        pltpu.async_copy(rows_bufs[b], out_hbm.at[pl.ds(obase, CHUNK)],
                         ssems[b])

    def wait_store(c, b):
        obase = out_base + c * CHUNK
        pltpu.make_async_copy(
            rows_bufs[b], out_hbm.at[pl.ds(obase, CHUNK)], ssems[b]
        ).wait()

    def outer_body(i, carry):
        all_handles = []
        for b in range(NBUF):
            c = i * NBUF + b

            @pl.when(i > 0)
            def _():
                wait_store(c, b)

            all_handles.append(fire_gathers(c, b))
        for b in range(NBUF):
            c = i * NBUF + b
            for h in all_handles[b]:
                h.wait()
            fire_store(c, b)
        return carry

    n_outer = CHUNKS_PER_W // NBUF
    lax.fori_loop(0, n_outer, outer_body, 0)
    for b in range(NBUF):
        c = (n_outer - 1) * NBUF + b
        wait_store(c, b)


def kernel(bond_types, table):
    idx2d = bond_types.reshape(E // IDX_MINOR, IDX_MINOR).astype(jnp.int32)
    mesh = plsc.VectorSubcoreMesh(core_axis_name="c", subcore_axis_name="s")
    kern = functools.partial(
        pl.kernel,
        out_type=jax.ShapeDtypeStruct((E, D), jnp.float32),
        mesh=mesh,
        scratch_types=[
            pltpu.VMEM_SHARED((NUM_ROWS, D), jnp.float32),
            pltpu.VMEM((G, IDX_MINOR), jnp.int32),
            pltpu.VMEM((G, IDX_MINOR), jnp.int32),
            pltpu.VMEM((CHUNK, D), jnp.float32),
            pltpu.VMEM((CHUNK, D), jnp.float32),
            pltpu.SemaphoreType.DMA,
            pltpu.SemaphoreType.DMA,
            pltpu.SemaphoreType.DMA,
            pltpu.SemaphoreType.DMA,
        ],
        compiler_params=pltpu.CompilerParams(use_tc_tiling_on_sc=False),
    )(_embed_body)
    return kern(idx2d, table)


# speedup vs baseline: 11.2534x; 1.0103x over previous
import functools

import jax
import jax.numpy as jnp
from jax import lax
from jax.experimental import pallas as pl
from jax.experimental.pallas import tpu as pltpu
from jax.experimental.pallas import tpu_sc as plsc

E = 800000
D = 64
NUM_ROWS = 5

NC = 2
NS = 16
NW = NC * NS

CHUNK = 1000
SUBS = [(0, 128), (128, 128), (256, 128), (384, 128),
        (512, 128), (640, 128), (768, 128), (896, 104)]
NBUF = 2
ROWS_PER_W = E // NW
CHUNKS_PER_W = ROWS_PER_W // CHUNK


def _embed_body(idx_hbm, table_hbm, out_hbm, table_sh,
                idx0, idx1, rows0, rows1, gsem0, gsem1, ssem0, ssem1):
    cid = lax.axis_index("c")
    sid = lax.axis_index("s")
    wid = cid * NS + sid
    out_base = wid * ROWS_PER_W

    @pl.when(sid == 0)
    def _():
        pltpu.sync_copy(table_hbm, table_sh)

    plsc.subcore_barrier()

    idx_bufs = (idx0, idx1)
    rows_bufs = (rows0, rows1)
    gsems = (gsem0, gsem1)
    ssems = (ssem0, ssem1)

    def fire_gathers(c, b):
        rbase = out_base + c * CHUNK
        pltpu.sync_copy(idx_hbm.at[pl.ds(rbase, CHUNK)], idx_bufs[b])
        handles = []
        for (off, ln) in SUBS:
            handles.append(pltpu.async_copy(
                table_sh.at[idx_bufs[b].at[pl.ds(off, ln)]],
                rows_bufs[b].at[pl.ds(off, ln)],
                gsems[b],
            ))
        return handles

    def fire_store(c, b):
        obase = out_base + c * CHUNK
        pltpu.async_copy(rows_bufs[b], out_hbm.at[pl.ds(obase, CHUNK)],
                         ssems[b])

    def wait_store(c, b):
        obase = out_base + c * CHUNK
        pltpu.make_async_copy(
            rows_bufs[b], out_hbm.at[pl.ds(obase, CHUNK)], ssems[b]
        ).wait()

    def outer_body(i, carry):
        all_handles = []
        for b in range(NBUF):
            c = i * NBUF + b

            @pl.when(i > 0)
            def _():
                wait_store(c, b)

            all_handles.append(fire_gathers(c, b))
        for b in range(NBUF):
            c = i * NBUF + b
            for h in all_handles[b]:
                h.wait()
            fire_store(c, b)
        return carry

    n_outer = CHUNKS_PER_W // NBUF
    lax.fori_loop(0, n_outer, outer_body, 0)
    for b in range(NBUF):
        c = (n_outer - 1) * NBUF + b
        wait_store(c, b)


def kernel(bond_types, table):
    idx1d = bond_types
    mesh = plsc.VectorSubcoreMesh(core_axis_name="c", subcore_axis_name="s")
    kern = functools.partial(
        pl.kernel,
        out_type=jax.ShapeDtypeStruct((E, D), jnp.float32),
        mesh=mesh,
        scratch_types=[
            pltpu.VMEM_SHARED((NUM_ROWS, D), jnp.float32),
            pltpu.VMEM((CHUNK,), jnp.int32),
            pltpu.VMEM((CHUNK,), jnp.int32),
            pltpu.VMEM((CHUNK, D), jnp.float32),
            pltpu.VMEM((CHUNK, D), jnp.float32),
            pltpu.SemaphoreType.DMA,
            pltpu.SemaphoreType.DMA,
            pltpu.SemaphoreType.DMA,
            pltpu.SemaphoreType.DMA,
        ],
        compiler_params=pltpu.CompilerParams(use_tc_tiling_on_sc=False),
    )(_embed_body)
    return kern(idx1d, table)
